# baseline (speedup 1.0000x reference)
import jax
import jax.numpy as jnp
from jax import lax
from jax.experimental import pallas as pl
from jax.experimental.pallas import tpu as pltpu
from jax.experimental.pallas import tpu_sc as plsc

B = 16384
D = 128
NC, NS, L = 2, 16, 16
NW = NC * NS
BPW = B // NW
CH = 128
NCH = BPW // CH
DEPTH = 2
IDEPTH = 3


def _sc_body(idx_h, ub_h, ib_h, ue_h, ie_h, te_h, out_h,
             uidx, iidx, tidx, urows, irows, ubv, ibv, scorev,
             gsem, isem2, tsem, bsem, isem, osem):
    wid = lax.axis_index("s") * NC + lax.axis_index("c")
    rowbase = wid * NCH

    def u_desc(c):
        return pltpu.make_async_copy(
            ue_h.at[uidx.at[c]], urows.at[c % DEPTH], gsem.at[c % DEPTH])

    def i_desc(c):
        return pltpu.make_async_copy(
            ie_h.at[iidx.at[c]], irows.at[c % IDEPTH], isem2.at[c % IDEPTH])

    def tadd_start(c):
        pltpu.async_copy(
            te_h.at[tidx.at[c]], irows.at[c % IDEPTH], tsem.at[c % IDEPTH], add=True)

    def tadd_wait(c):
        pltpu.make_async_copy(
            te_h.at[tidx.at[c]], irows.at[c % IDEPTH], tsem.at[c % IDEPTH]).wait()

    def bias_descs(c):
        return (
            pltpu.make_async_copy(ub_h.at[uidx.at[c]], ubv.at[c], bsem),
            pltpu.make_async_copy(ib_h.at[iidx.at[c]], ibv.at[c], bsem),
        )

    idx_stage = (
        pltpu.make_async_copy(idx_h.at[0].at[pl.ds(rowbase, NCH)], uidx, isem.at[0]),
        pltpu.make_async_copy(idx_h.at[1].at[pl.ds(rowbase, NCH)], iidx, isem.at[1]),
        pltpu.make_async_copy(idx_h.at[2].at[pl.ds(rowbase, NCH)], tidx, isem.at[2]),
    )
    for d in idx_stage:
        d.start()
    idx_stage[0].wait()
    for c in range(DEPTH):
        u_desc(c).start()
    for c in range(NCH):
        bias_descs(c)[0].start()
    idx_stage[1].wait()
    for c in range(min(IDEPTH, NCH)):
        i_desc(c).start()
    for c in range(NCH):
        bias_descs(c)[1].start()
    idx_stage[2].wait()
    i_desc(0).wait()
    tadd_start(0)

    for i in range(NCH):
        slot = i % DEPTH
        islot = i % IDEPTH
        if i + 1 < NCH:
            i_desc(i + 1).wait()
            tadd_start(i + 1)
        u_desc(i).wait()
        tadd_wait(i)
        for d in bias_descs(i):
            d.wait()

        def group(g, carry):
            lane = lax.iota(jnp.int32, L)

            def row(rr, acc):
                r = g * L + rr

                def term(k):
                    return urows[slot, r, pl.ds(k * L, L)] * irows[islot, r, pl.ds(k * L, L)]

                dv0 = term(0)
                dv1 = term(1)
                for k in range(2, D // L, 2):
                    dv0 = dv0 + term(k)
                    dv1 = dv1 + term(k + 1)
                return jnp.where(lane == rr, jnp.sum(dv0 + dv1), acc)

            acc = lax.fori_loop(0, L, row, jnp.zeros((L,), jnp.float32))
            scorev[i, pl.ds(g * L, L)] = (
                acc + ubv[i, pl.ds(g * L, L)] + ibv[i, pl.ds(g * L, L)]
            )
            return carry

        lax.fori_loop(0, CH // L, group, 0)
        if i + DEPTH < NCH:
            u_desc(i + DEPTH).start()
        if i + IDEPTH < NCH:
            i_desc(i + IDEPTH).start()

    out_copy = pltpu.make_async_copy(scorev, out_h.at[pl.ds(rowbase, NCH)], osem)
    out_copy.start()
    out_copy.wait()


def kernel(user, item, it_in, it_off, u_bias_w, i_bias_w, u_embed_w, i_embed_w, t_embed_w):
    del it_off
    ub = u_bias_w.reshape(-1)
    ib = i_bias_w.reshape(-1)
    idx3 = jnp.stack([user, item, it_in]).reshape(3, B // CH, CH)
    mesh = plsc.VectorSubcoreMesh(core_axis_name="c", subcore_axis_name="s")
    run = pl.kernel(
        _sc_body,
        out_type=jax.ShapeDtypeStruct((B // CH, CH), jnp.float32),
        mesh=mesh,
        compiler_params=pltpu.CompilerParams(needs_layout_passes=False),
        scratch_types=[
            pltpu.VMEM((NCH, CH), jnp.int32),
            pltpu.VMEM((NCH, CH), jnp.int32),
            pltpu.VMEM((NCH, CH), jnp.int32),
            pltpu.VMEM((DEPTH, CH, D), jnp.float32),
            pltpu.VMEM((IDEPTH, CH, D), jnp.float32),
            pltpu.VMEM((NCH, CH), jnp.float32),
            pltpu.VMEM((NCH, CH), jnp.float32),
            pltpu.VMEM((NCH, CH), jnp.float32),
            pltpu.SemaphoreType.DMA((DEPTH,)),
            pltpu.SemaphoreType.DMA((IDEPTH,)),
            pltpu.SemaphoreType.DMA((IDEPTH,)),
            pltpu.SemaphoreType.DMA,
            pltpu.SemaphoreType.DMA((3,)),
            pltpu.SemaphoreType.DMA,
        ],
    )
    out2 = run(idx3, ub, ib, u_embed_w, i_embed_w, t_embed_w)
    return out2.reshape(B)


# speedup vs baseline: 1.0411x; 1.0411x over previous
import jax
import jax.numpy as jnp
from jax import lax
from jax.experimental import pallas as pl
from jax.experimental.pallas import tpu as pltpu
from jax.experimental.pallas import tpu_sc as plsc

B = 16384
D = 128
NC, NS, L = 2, 16, 16
NW = NC * NS
BPW = B // NW
CH = 128
NCH = BPW // CH
DEPTH = 2
IDEPTH = 3


def _sc_body(user_h, item_h, tag_h, ub_h, ib_h, ue_h, ie_h, te_h, out_h,
             uidx, iidx, tidx, urows, irows, ubv, ibv, scorev,
             gsem, isem2, tsem, bsem, isem, osem):
    wid = lax.axis_index("s") * NC + lax.axis_index("c")
    rowbase = wid * NCH

    def u_desc(c):
        return pltpu.make_async_copy(
            ue_h.at[uidx.at[c]], urows.at[c % DEPTH], gsem.at[c % DEPTH])

    def i_desc(c):
        return pltpu.make_async_copy(
            ie_h.at[iidx.at[c]], irows.at[c % IDEPTH], isem2.at[c % IDEPTH])

    def tadd_start(c):
        pltpu.async_copy(
            te_h.at[tidx.at[c]], irows.at[c % IDEPTH], tsem.at[c % IDEPTH], add=True)

    def tadd_wait(c):
        pltpu.make_async_copy(
            te_h.at[tidx.at[c]], irows.at[c % IDEPTH], tsem.at[c % IDEPTH]).wait()

    def bias_descs(c):
        return (
            pltpu.make_async_copy(ub_h.at[uidx.at[c]], ubv.at[c], bsem),
            pltpu.make_async_copy(ib_h.at[iidx.at[c]], ibv.at[c], bsem),
        )

    idx_stage = (
        pltpu.make_async_copy(user_h.at[pl.ds(rowbase, NCH)], uidx, isem.at[0]),
        pltpu.make_async_copy(item_h.at[pl.ds(rowbase, NCH)], iidx, isem.at[1]),
        pltpu.make_async_copy(tag_h.at[pl.ds(rowbase, NCH)], tidx, isem.at[2]),
    )
    for d in idx_stage:
        d.start()
    idx_stage[0].wait()
    for c in range(DEPTH):
        u_desc(c).start()
    for c in range(NCH):
        bias_descs(c)[0].start()
    idx_stage[1].wait()
    for c in range(min(IDEPTH, NCH)):
        i_desc(c).start()
    for c in range(NCH):
        bias_descs(c)[1].start()
    idx_stage[2].wait()
    i_desc(0).wait()
    tadd_start(0)

    for i in range(NCH):
        slot = i % DEPTH
        islot = i % IDEPTH
        if i + 1 < NCH:
            i_desc(i + 1).wait()
            tadd_start(i + 1)
        u_desc(i).wait()
        tadd_wait(i)
        for d in bias_descs(i):
            d.wait()

        def group(g, carry):
            lane = lax.iota(jnp.int32, L)

            def row(rr, acc):
                r = g * L + rr

                def term(k):
                    return urows[slot, r, pl.ds(k * L, L)] * irows[islot, r, pl.ds(k * L, L)]

                dv0 = term(0)
                dv1 = term(1)
                for k in range(2, D // L, 2):
                    dv0 = dv0 + term(k)
                    dv1 = dv1 + term(k + 1)
                return jnp.where(lane == rr, jnp.sum(dv0 + dv1), acc)

            acc = lax.fori_loop(0, L, row, jnp.zeros((L,), jnp.float32))
            scorev[i, pl.ds(g * L, L)] = (
                acc + ubv[i, pl.ds(g * L, L)] + ibv[i, pl.ds(g * L, L)]
            )
            return carry

        lax.fori_loop(0, CH // L, group, 0)
        if i + DEPTH < NCH:
            u_desc(i + DEPTH).start()
        if i + IDEPTH < NCH:
            i_desc(i + IDEPTH).start()

    out_copy = pltpu.make_async_copy(scorev, out_h.at[pl.ds(rowbase, NCH)], osem)
    out_copy.start()
    out_copy.wait()


def kernel(user, item, it_in, it_off, u_bias_w, i_bias_w, u_embed_w, i_embed_w, t_embed_w):
    del it_off
    ub = u_bias_w.reshape(-1)
    ib = i_bias_w.reshape(-1)
    user2 = user.reshape(B // CH, CH)
    item2 = item.reshape(B // CH, CH)
    tag2 = it_in.reshape(B // CH, CH)
    mesh = plsc.VectorSubcoreMesh(core_axis_name="c", subcore_axis_name="s")
    run = pl.kernel(
        _sc_body,
        out_type=jax.ShapeDtypeStruct((B // CH, CH), jnp.float32),
        mesh=mesh,
        compiler_params=pltpu.CompilerParams(needs_layout_passes=False, use_tc_tiling_on_sc=False),
        scratch_types=[
            pltpu.VMEM((NCH, CH), jnp.int32),
            pltpu.VMEM((NCH, CH), jnp.int32),
            pltpu.VMEM((NCH, CH), jnp.int32),
            pltpu.VMEM((DEPTH, CH, D), jnp.float32),
            pltpu.VMEM((IDEPTH, CH, D), jnp.float32),
            pltpu.VMEM((NCH, CH), jnp.float32),
            pltpu.VMEM((NCH, CH), jnp.float32),
            pltpu.VMEM((NCH, CH), jnp.float32),
            pltpu.SemaphoreType.DMA((DEPTH,)),
            pltpu.SemaphoreType.DMA((IDEPTH,)),
            pltpu.SemaphoreType.DMA((IDEPTH,)),
            pltpu.SemaphoreType.DMA,
            pltpu.SemaphoreType.DMA((3,)),
            pltpu.SemaphoreType.DMA,
        ],
    )
    out2 = run(user2, item2, tag2, ub, ib, u_embed_w, i_embed_w, t_embed_w)
    return out2.reshape(B)
